# Initial kernel scaffold; baseline (speedup 1.0000x reference)
import jax
import jax.numpy as jnp
from jax.experimental import pallas as pl


def kernel(x, M):
    raise NotImplementedError("write your pallas kernel here")


# speedup vs baseline: 11.6195x; 11.6195x over previous
import functools

import jax
import jax.numpy as jnp
from jax import lax
from jax.experimental import pallas as pl
from jax.experimental.pallas import tpu as pltpu
from jax.experimental.pallas import tpu_sc as plsc

B = 16
NC, NS = 2, 16
NW = NC * NS
TRIS_PER_W = 3200
T_PAD = NW * TRIS_PER_W
CHUNK = 128
N_CHUNKS = TRIS_PER_W // CHUNK
ROW = 3 * B


def _sc_volume_partials(xt, m0, m1, m2):
    mesh = plsc.VectorSubcoreMesh(core_axis_name="c", subcore_axis_name="s")

    @functools.partial(
        pl.kernel,
        mesh=mesh,
        out_type=jax.ShapeDtypeStruct((NW, B), jnp.float32),
        compiler_params=pltpu.CompilerParams(use_tc_tiling_on_sc=False),
        scratch_types=[
            pltpu.VMEM((CHUNK,), jnp.int32),
            pltpu.VMEM((CHUNK,), jnp.int32),
            pltpu.VMEM((CHUNK,), jnp.int32),
            pltpu.VMEM((CHUNK, ROW), jnp.float32),
            pltpu.VMEM((CHUNK, ROW), jnp.float32),
            pltpu.VMEM((CHUNK, ROW), jnp.float32),
            pltpu.VMEM((B,), jnp.float32),
            pltpu.SemaphoreType.DMA,
        ],
    )
    def k(xt_hbm, m0_hbm, m1_hbm, m2_hbm, out_hbm,
          i0, i1, i2, r0, r1, r2, accv, sem):
        wid = lax.axis_index("s") * NC + lax.axis_index("c")
        base = wid * TRIS_PER_W

        def chunk_body(ci, acc):
            off = base + ci * CHUNK
            pltpu.sync_copy(m0_hbm.at[pl.ds(off, CHUNK)], i0)
            pltpu.sync_copy(m1_hbm.at[pl.ds(off, CHUNK)], i1)
            pltpu.sync_copy(m2_hbm.at[pl.ds(off, CHUNK)], i2)
            c0 = pltpu.async_copy(xt_hbm.at[i0], r0, sem)
            c1 = pltpu.async_copy(xt_hbm.at[i1], r1, sem)
            c2 = pltpu.async_copy(xt_hbm.at[i2], r2, sem)
            c0.wait()
            c1.wait()
            c2.wait()

            def tri_body(t, a):
                a1 = r0[t, pl.ds(0, B)]
                a2 = r0[t, pl.ds(B, B)]
                a3 = r0[t, pl.ds(2 * B, B)]
                b1 = r1[t, pl.ds(0, B)]
                b2 = r1[t, pl.ds(B, B)]
                b3 = r1[t, pl.ds(2 * B, B)]
                d1 = r2[t, pl.ds(0, B)]
                d2 = r2[t, pl.ds(B, B)]
                d3 = r2[t, pl.ds(2 * B, B)]
                det = (a1 * (b2 * d3 - b3 * d2)
                       - a2 * (b1 * d3 - b3 * d1)
                       + a3 * (b1 * d2 - b2 * d1))
                return a + jnp.abs(det)

            return lax.fori_loop(0, CHUNK, tri_body, acc, unroll=4)

        acc = lax.fori_loop(0, N_CHUNKS, chunk_body,
                            jnp.zeros((B,), jnp.float32))
        accv[...] = acc
        pltpu.sync_copy(accv, out_hbm.at[wid])

    return k(xt, m0, m1, m2)


_XBLK = 2048


def _normalize(x, partials):
    cols = x.shape[1]
    grid = pl.cdiv(cols, _XBLK)

    def body(p_ref, x_ref, o_ref):
        tot = jnp.sum(p_ref[...], axis=0)
        vol = tot * (1.0 / 6.0)
        inv = jnp.exp(jnp.log(vol) * (-1.0 / 3.0))
        o_ref[...] = x_ref[...] * inv[:, None]

    return pl.pallas_call(
        body,
        grid=(grid,),
        in_specs=[
            pl.BlockSpec((NW, B), lambda i: (0, 0)),
            pl.BlockSpec((B, _XBLK), lambda i: (0, i)),
        ],
        out_specs=pl.BlockSpec((B, _XBLK), lambda i: (0, i)),
        out_shape=jax.ShapeDtypeStruct(x.shape, x.dtype),
    )(partials, x)


def kernel(x, M):
    n_verts = x.shape[1] // 3
    xt = x.reshape(B, n_verts, 3).transpose(1, 2, 0).reshape(n_verts, ROW)
    Mi = M.astype(jnp.int32)
    pad = T_PAD - Mi.shape[0]
    Mp = jnp.concatenate([Mi, jnp.zeros((pad, 3), jnp.int32)], axis=0)
    Mt = Mp.T
    m0, m1, m2 = Mt[0], Mt[1], Mt[2]
    partials = _sc_volume_partials(xt, m0, m1, m2)
    return _normalize(x, partials)
